# baseline (speedup 1.0000x reference)
import functools
import math

import jax
import jax.numpy as jnp
from jax import lax
from jax.experimental import pallas as pl
from jax.experimental.pallas import tpu as pltpu
from jax.experimental.pallas import tpu_sc as plsc

N = 10000
E = 320000
F_IN = 128
H = 64
C = 40
BN_EPS = 1e-5
BN_INV = 1.0 / math.sqrt(1.0 + BN_EPS)

NTILES = 32
CH = 80
TPT = E // NTILES
NCH = TPT // CH
ROWS2D = E // CH

R = 2048
GRID = 5
NPAD = R * GRID

_mesh = plsc.VectorSubcoreMesh(core_axis_name="c", subcore_axis_name="s")
_sc_params = pltpu.CompilerParams(use_tc_tiling_on_sc=False)


@functools.partial(
    pl.kernel,
    mesh=_mesh,
    out_type=jax.ShapeDtypeStruct((2, NPAD), jnp.float32),
    compiler_params=_sc_params,
    scratch_types=[
        pltpu.VMEM((NCH, CH), jnp.int32),
        pltpu.VMEM((NCH, CH), jnp.float32),
        pltpu.VMEM((R,), jnp.float32),
        pltpu.VMEM_SHARED((NPAD,), jnp.float32),
    ],
)
def _sc_deg(ei_hbm, w_hbm, out_hbm, dst_v, w_v, zbuf, deg_sh):
    c = lax.axis_index("c")
    s = lax.axis_index("s")
    wid = c * 16 + s
    pltpu.sync_copy(ei_hbm.at[1, pl.ds(wid * NCH, NCH)], dst_v)
    pltpu.sync_copy(w_hbm.at[pl.ds(wid * NCH, NCH)], w_v)

    @pl.when(s == 0)
    def _():
        def zb(k, carry):
            zbuf[pl.ds(k * 16, 16)] = jnp.zeros((16,), jnp.float32)
            return carry
        lax.fori_loop(0, R // 16, zb, 0)
        for t in range(GRID):
            pltpu.sync_copy(zbuf, deg_sh.at[pl.ds(t * R, R)])

    plsc.subcore_barrier()

    def body(j, carry):
        pltpu.sync_copy(w_v.at[j], deg_sh.at[dst_v.at[j]], add=True)
        return carry
    lax.fori_loop(0, NCH, body, 0)

    plsc.subcore_barrier()
    pltpu.sync_copy(deg_sh.at[pl.ds(s * (NPAD // 16), NPAD // 16)],
                    out_hbm.at[c, pl.ds(s * (NPAD // 16), NPAD // 16)])


@functools.partial(
    pl.kernel,
    mesh=_mesh,
    out_type=jax.ShapeDtypeStruct((2, N, H), jnp.float32),
    compiler_params=_sc_params,
    scratch_types=[
        pltpu.VMEM((NCH, CH), jnp.int32),
        pltpu.VMEM((NCH, CH), jnp.int32),
        pltpu.VMEM((NCH, CH), jnp.float32),
        [pltpu.VMEM((CH, H), jnp.float32)] * 6,
        pltpu.VMEM((125, H), jnp.float32),
        pltpu.VMEM_SHARED((N, H), jnp.float32),
        [pltpu.SemaphoreType.DMA] * 6,
        [pltpu.SemaphoreType.DMA] * 6,
    ],
)
def _sc_conv(g_hbm, ei_hbm, w_hbm, out_hbm,
             src_v, dst_v, w_v, bufs, zv, acc_sh, gsems, ssems):
    c = lax.axis_index("c")
    s = lax.axis_index("s")
    wid = c * 16 + s
    pltpu.sync_copy(ei_hbm.at[0, pl.ds(wid * NCH, NCH)], src_v)
    pltpu.sync_copy(ei_hbm.at[1, pl.ds(wid * NCH, NCH)], dst_v)
    pltpu.sync_copy(w_hbm.at[pl.ds(wid * NCH, NCH)], w_v)

    def zb(i, carry):
        for cc in range(H // 16):
            zv[i, pl.ds(cc * 16, 16)] = jnp.zeros((16,), jnp.float32)
        return carry
    lax.fori_loop(0, 125, zb, 0)
    for t in range(5):
        pltpu.sync_copy(zv, acc_sh.at[pl.ds(s * 625 + t * 125, 125)])

    plsc.subcore_barrier()

    def _start_gather(j, b):
        pltpu.async_copy(g_hbm.at[src_v.at[j]], bufs[b], gsems[b])

    def _wait_gather(j, b):
        pltpu.make_async_copy(g_hbm.at[src_v.at[j]], bufs[b], gsems[b]).wait()

    def _start_scatter(j, b):
        pltpu.async_copy(bufs[b], acc_sh.at[dst_v.at[j]], ssems[b], add=True)

    def _wait_scatter(j, b):
        pltpu.make_async_copy(
            bufs[b], acc_sh.at[dst_v.at[j]], ssems[b]).wait()

    def _mul(j, b):
        rows = bufs[b]
        _LANE = lax.iota(jnp.int32, 16)

        @plsc.parallel_loop(0, CH // 16, unroll=2)
        def mul(g):
            i0 = g * 16
            wvec = w_v[j, pl.ds(i0, 16)]
            for k in range(16):
                wv = wvec[k]
                for cc in range(H // 16):
                    sl = pl.ds(cc * 16, 16)
                    rows[i0 + k, sl] = rows[i0 + k, sl] * wv

    NBUF = 6
    LA = 3

    def _step(j, b):
        _wait_gather(j, b)
        _mul(j, b)
        _start_scatter(j, b)

        @pl.when(j + LA < NCH)
        def _():
            b2 = (b + LA) % NBUF

            @pl.when(j >= LA)
            def _():
                _wait_scatter(j - (NBUF - LA), b2)
            _start_gather(j + LA, b2)

    for jp in range(LA):
        _start_gather(jp, jp)

    def body(jj, carry):
        for b in range(NBUF):
            _step(jj * NBUF + b, b)
        return carry
    lax.fori_loop(0, NCH // NBUF, body, 0)
    for jt in range((NCH // NBUF) * NBUF, NCH):
        _step(jt, jt % NBUF)
    for jt in range(NCH - NBUF, NCH):
        _wait_scatter(jt, jt % NBUF)

    plsc.subcore_barrier()
    pltpu.sync_copy(acc_sh.at[pl.ds(s * 625, 625)],
                    out_hbm.at[c, pl.ds(s * 625, 625)])


def _leaky(v):
    return jnp.where(v >= 0.0, v, 0.01 * v)


def _dinv_from(degp_ref):
    deg = degp_ref[0] + degp_ref[1] + 1.0
    return jnp.where(deg > 0.0, lax.rsqrt(deg), 0.0)


def _tc1_body(x_ref, wft_ref, bf_ref, g1w_ref, b1w_ref, wc1t_ref,
              h0_ref, hw1_ref):
    h = jnp.dot(x_ref[...], wft_ref[...], preferred_element_type=jnp.float32)
    h = _leaky(h + bf_ref[...])
    h0 = h * (g1w_ref[...] * BN_INV) + b1w_ref[...]
    h0_ref[...] = h0
    hw1_ref[...] = jnp.dot(h0, wc1t_ref[...], preferred_element_type=jnp.float32)


def _tcs_body(hw_ref, degp_ref, g_ref):
    g_ref[...] = hw_ref[...] * _dinv_from(degp_ref)


def _tc2_body(accp_ref, g1_ref, degp_ref, bc1_ref, bng_ref, bnb_ref, wc2t_ref,
              h1_ref, g2_ref):
    dinv = _dinv_from(degp_ref)
    pre = dinv * (accp_ref[0] + accp_ref[1] + g1_ref[...]) + bc1_ref[...]
    h1 = _leaky(pre) * (bng_ref[...] * BN_INV) + bnb_ref[...]
    h1_ref[...] = h1
    hw2 = jnp.dot(h1, wc2t_ref[...], preferred_element_type=jnp.float32)
    g2_ref[...] = hw2 * dinv


def _tc3_body(accp_ref, g2_ref, degp_ref, bc2_ref, bng_ref, bnb_ref,
              h0_ref, h1_ref, w0t_ref, w1t_ref, w2t_ref, bpad_ref, out_ref):
    dinv = _dinv_from(degp_ref)
    pre = dinv * (accp_ref[0] + accp_ref[1] + g2_ref[...]) + bc2_ref[...]
    h2 = _leaky(pre) * (bng_ref[...] * BN_INV) + bnb_ref[...]
    lg = (jnp.dot(h0_ref[...], w0t_ref[...], preferred_element_type=jnp.float32)
          + jnp.dot(h1_ref[...], w1t_ref[...], preferred_element_type=jnp.float32)
          + jnp.dot(h2, w2t_ref[...], preferred_element_type=jnp.float32)
          + bpad_ref[...])
    m = jnp.max(lg, axis=-1, keepdims=True)
    sh = lg - m
    lse = jnp.log(jnp.sum(jnp.exp(sh), axis=-1, keepdims=True))
    out_ref[...] = (sh - lse)[:, :C]


def _row_spec(cols):
    return pl.BlockSpec((R, cols), lambda i: (i, 0))


def _full_spec(shape):
    nd = len(shape)
    return pl.BlockSpec(shape, lambda i, _nd=nd: (0,) * _nd)


_degp_spec = pl.BlockSpec((2, R, 1), lambda i: (0, i, 0))
_accp_spec = pl.BlockSpec((2, R, H), lambda i: (0, i, 0))

_tc1 = pl.pallas_call(
    _tc1_body,
    grid=(GRID,),
    in_specs=[
        _row_spec(F_IN),
        _full_spec((F_IN, H)),
        _full_spec((1, H)),
        _full_spec((1, H)),
        _full_spec((1, H)),
        _full_spec((H, H)),
    ],
    out_specs=[_row_spec(H), _row_spec(H)],
    out_shape=[
        jax.ShapeDtypeStruct((N, H), jnp.float32),
        jax.ShapeDtypeStruct((N, H), jnp.float32),
    ],
)

_tcs = pl.pallas_call(
    _tcs_body,
    grid=(GRID,),
    in_specs=[_row_spec(H), _degp_spec],
    out_specs=[_row_spec(H)],
    out_shape=[jax.ShapeDtypeStruct((N, H), jnp.float32)],
)

_tc2 = pl.pallas_call(
    _tc2_body,
    grid=(GRID,),
    in_specs=[
        _accp_spec,
        _row_spec(H),
        _degp_spec,
        _full_spec((1, H)),
        _full_spec((1, H)),
        _full_spec((1, H)),
        _full_spec((H, H)),
    ],
    out_specs=[_row_spec(H), _row_spec(H)],
    out_shape=[
        jax.ShapeDtypeStruct((N, H), jnp.float32),
        jax.ShapeDtypeStruct((N, H), jnp.float32),
    ],
)

_tc3 = pl.pallas_call(
    _tc3_body,
    grid=(GRID,),
    in_specs=[
        _accp_spec,
        _row_spec(H),
        _degp_spec,
        _full_spec((1, H)),
        _full_spec((1, H)),
        _full_spec((1, H)),
        _row_spec(H),
        _row_spec(H),
        _full_spec((H, 128)),
        _full_spec((H, 128)),
        _full_spec((H, 128)),
        _full_spec((1, 128)),
    ],
    out_specs=[_row_spec(C)],
    out_shape=[jax.ShapeDtypeStruct((N, C), jnp.float32)],
)


def kernel(x, edge_index, edge_weight, W_first, b_first, bn1_g, bn1_b,
           Wc1, bc1, bng1, bnb1, Wc2, bc2, bng2, bnb2, W_out, b_out):
    ei = edge_index.reshape(2, ROWS2D, CH)
    w2d = edge_weight.reshape(ROWS2D, CH)

    degp = _sc_deg(ei, w2d)
    degp3 = degp.reshape(2, NPAD, 1)

    wft = W_first.T
    wc1t = Wc1.T
    wc2t = Wc2.T
    row = lambda v: v.reshape(1, H)

    h0, hw1 = _tc1(x, wft, row(b_first), row(bn1_g), row(bn1_b), wc1t)
    (g1,) = _tcs(hw1, degp3)
    acc1 = _sc_conv(g1, ei, w2d)
    h1, g2 = _tc2(acc1, g1, degp3, row(bc1), row(bng1), row(bnb1), wc2t)
    acc2 = _sc_conv(g2, ei, w2d)

    wot = jnp.pad(W_out.T, ((0, 0), (0, 128 - C)))
    bpad = jnp.pad(b_out, (0, 128 - C), constant_values=-1e30).reshape(1, 128)
    (out,) = _tc3(acc2, g2, degp3, row(bc2), row(bng2), row(bnb2),
                  h0, h1, wot[:H], wot[H:2 * H], wot[2 * H:], bpad)
    return out


# speedup vs baseline: 1.0035x; 1.0035x over previous
import functools
import math

import jax
import jax.numpy as jnp
from jax import lax
from jax.experimental import pallas as pl
from jax.experimental.pallas import tpu as pltpu
from jax.experimental.pallas import tpu_sc as plsc

N = 10000
E = 320000
F_IN = 128
H = 64
C = 40
BN_EPS = 1e-5
BN_INV = 1.0 / math.sqrt(1.0 + BN_EPS)

NTILES = 32
CH = 80
TPT = E // NTILES
NCH = TPT // CH
ROWS2D = E // CH

R = 2048
GRID = 5
NPAD = R * GRID

_mesh = plsc.VectorSubcoreMesh(core_axis_name="c", subcore_axis_name="s")
_sc_params = pltpu.CompilerParams(use_tc_tiling_on_sc=False)


@functools.partial(
    pl.kernel,
    mesh=_mesh,
    out_type=jax.ShapeDtypeStruct((2, NPAD), jnp.float32),
    compiler_params=_sc_params,
    scratch_types=[
        pltpu.VMEM((NCH, CH), jnp.int32),
        pltpu.VMEM((NCH, CH), jnp.float32),
        pltpu.VMEM((R,), jnp.float32),
        pltpu.VMEM_SHARED((NPAD,), jnp.float32),
    ],
)
def _sc_deg(ei_hbm, w_hbm, out_hbm, dst_v, w_v, zbuf, deg_sh):
    c = lax.axis_index("c")
    s = lax.axis_index("s")
    wid = c * 16 + s
    pltpu.sync_copy(ei_hbm.at[1, pl.ds(wid * NCH, NCH)], dst_v)
    pltpu.sync_copy(w_hbm.at[pl.ds(wid * NCH, NCH)], w_v)

    @pl.when(s == 0)
    def _():
        def zb(k, carry):
            zbuf[pl.ds(k * 16, 16)] = jnp.zeros((16,), jnp.float32)
            return carry
        lax.fori_loop(0, R // 16, zb, 0)
        for t in range(GRID):
            pltpu.sync_copy(zbuf, deg_sh.at[pl.ds(t * R, R)])

    plsc.subcore_barrier()

    def body(j, carry):
        pltpu.sync_copy(w_v.at[j], deg_sh.at[dst_v.at[j]], add=True)
        return carry
    lax.fori_loop(0, NCH, body, 0)

    plsc.subcore_barrier()
    pltpu.sync_copy(deg_sh.at[pl.ds(s * (NPAD // 16), NPAD // 16)],
                    out_hbm.at[c, pl.ds(s * (NPAD // 16), NPAD // 16)])


@functools.partial(
    pl.kernel,
    mesh=_mesh,
    out_type=jax.ShapeDtypeStruct((2, N, H), jnp.float32),
    compiler_params=_sc_params,
    scratch_types=[
        pltpu.VMEM((NCH, CH), jnp.int32),
        pltpu.VMEM((NCH, CH), jnp.int32),
        pltpu.VMEM((NCH, CH), jnp.float32),
        [pltpu.VMEM((CH, H), jnp.float32)] * 6,
        pltpu.VMEM((125, H), jnp.float32),
        pltpu.VMEM_SHARED((N, H), jnp.float32),
        [pltpu.SemaphoreType.DMA] * 6,
        [pltpu.SemaphoreType.DMA] * 6,
    ],
)
def _sc_conv(g_hbm, ei_hbm, w_hbm, out_hbm,
             src_v, dst_v, w_v, bufs, zv, acc_sh, gsems, ssems):
    c = lax.axis_index("c")
    s = lax.axis_index("s")
    wid = c * 16 + s
    pltpu.sync_copy(ei_hbm.at[0, pl.ds(wid * NCH, NCH)], src_v)
    pltpu.sync_copy(ei_hbm.at[1, pl.ds(wid * NCH, NCH)], dst_v)
    pltpu.sync_copy(w_hbm.at[pl.ds(wid * NCH, NCH)], w_v)

    def zb(i, carry):
        for cc in range(H // 16):
            zv[i, pl.ds(cc * 16, 16)] = jnp.zeros((16,), jnp.float32)
        return carry
    lax.fori_loop(0, 125, zb, 0)
    for t in range(5):
        pltpu.sync_copy(zv, acc_sh.at[pl.ds(s * 625 + t * 125, 125)])

    plsc.subcore_barrier()

    def _start_gather(j, b):
        pltpu.async_copy(g_hbm.at[src_v.at[j]], bufs[b], gsems[b])

    def _wait_gather(j, b):
        pltpu.make_async_copy(g_hbm.at[src_v.at[j]], bufs[b], gsems[b]).wait()

    def _start_scatter(j, b):
        pltpu.async_copy(bufs[b], acc_sh.at[dst_v.at[j]], ssems[b], add=True)

    def _wait_scatter(j, b):
        pltpu.make_async_copy(
            bufs[b], acc_sh.at[dst_v.at[j]], ssems[b]).wait()

    def _mul(j, b):
        rows = bufs[b]

        @plsc.parallel_loop(0, CH // 16, unroll=2)
        def mul(g):
            i0 = g * 16
            wvec = w_v[j, pl.ds(i0, 16)]
            for k in range(16):
                wv = wvec[k]
                for cc in range(H // 16):
                    sl = pl.ds(cc * 16, 16)
                    rows[i0 + k, sl] = rows[i0 + k, sl] * wv

    NBUF = 6
    LA = 3

    def _step(j, b):
        _wait_gather(j, b)
        _mul(j, b)
        _start_scatter(j, b)

        @pl.when(j + LA < NCH)
        def _():
            b2 = (b + LA) % NBUF

            @pl.when(j >= LA)
            def _():
                _wait_scatter(j - (NBUF - LA), b2)
            _start_gather(j + LA, b2)

    for jp in range(LA):
        _start_gather(jp, jp)

    def body(jj, carry):
        for b in range(NBUF):
            _step(jj * NBUF + b, b)
        return carry
    lax.fori_loop(0, NCH // NBUF, body, 0)
    for jt in range((NCH // NBUF) * NBUF, NCH):
        _step(jt, jt % NBUF)
    for jt in range(NCH - NBUF, NCH):
        _wait_scatter(jt, jt % NBUF)

    plsc.subcore_barrier()
    pltpu.sync_copy(acc_sh.at[pl.ds(s * 625, 625)],
                    out_hbm.at[c, pl.ds(s * 625, 625)])


def _leaky(v):
    return jnp.where(v >= 0.0, v, 0.01 * v)


def _dinv_from(degp_ref):
    deg = degp_ref[0] + degp_ref[1] + 1.0
    return jnp.where(deg > 0.0, lax.rsqrt(deg), 0.0)


def _tc1_body(x_ref, wft_ref, bf_ref, g1w_ref, b1w_ref, wc1t_ref,
              h0_ref, hw1_ref):
    h = jnp.dot(x_ref[...], wft_ref[...], preferred_element_type=jnp.float32)
    h = _leaky(h + bf_ref[...])
    h0 = h * (g1w_ref[...] * BN_INV) + b1w_ref[...]
    h0_ref[...] = h0
    hw1_ref[...] = jnp.dot(h0, wc1t_ref[...], preferred_element_type=jnp.float32)


def _tcs_body(hw_ref, degp_ref, g_ref):
    g_ref[...] = hw_ref[...] * _dinv_from(degp_ref)


def _tc2_body(accp_ref, g1_ref, degp_ref, bc1_ref, bng_ref, bnb_ref, wc2t_ref,
              h1_ref, g2_ref):
    dinv = _dinv_from(degp_ref)
    pre = dinv * (accp_ref[0] + accp_ref[1] + g1_ref[...]) + bc1_ref[...]
    h1 = _leaky(pre) * (bng_ref[...] * BN_INV) + bnb_ref[...]
    h1_ref[...] = h1
    hw2 = jnp.dot(h1, wc2t_ref[...], preferred_element_type=jnp.float32)
    g2_ref[...] = hw2 * dinv


def _tc3_body(accp_ref, g2_ref, degp_ref, bc2_ref, bng_ref, bnb_ref,
              h0_ref, h1_ref, w0t_ref, w1t_ref, w2t_ref, bpad_ref, out_ref):
    dinv = _dinv_from(degp_ref)
    pre = dinv * (accp_ref[0] + accp_ref[1] + g2_ref[...]) + bc2_ref[...]
    h2 = _leaky(pre) * (bng_ref[...] * BN_INV) + bnb_ref[...]
    lg = (jnp.dot(h0_ref[...], w0t_ref[...], preferred_element_type=jnp.float32)
          + jnp.dot(h1_ref[...], w1t_ref[...], preferred_element_type=jnp.float32)
          + jnp.dot(h2, w2t_ref[...], preferred_element_type=jnp.float32)
          + bpad_ref[...])
    m = jnp.max(lg, axis=-1, keepdims=True)
    sh = lg - m
    lse = jnp.log(jnp.sum(jnp.exp(sh), axis=-1, keepdims=True))
    out_ref[...] = (sh - lse)[:, :C]


def _row_spec(cols):
    return pl.BlockSpec((R, cols), lambda i: (i, 0))


def _full_spec(shape):
    nd = len(shape)
    return pl.BlockSpec(shape, lambda i, _nd=nd: (0,) * _nd)


_degp_spec = pl.BlockSpec((2, R, 1), lambda i: (0, i, 0))
_accp_spec = pl.BlockSpec((2, R, H), lambda i: (0, i, 0))

_tc1 = pl.pallas_call(
    _tc1_body,
    grid=(GRID,),
    in_specs=[
        _row_spec(F_IN),
        _full_spec((F_IN, H)),
        _full_spec((1, H)),
        _full_spec((1, H)),
        _full_spec((1, H)),
        _full_spec((H, H)),
    ],
    out_specs=[_row_spec(H), _row_spec(H)],
    out_shape=[
        jax.ShapeDtypeStruct((N, H), jnp.float32),
        jax.ShapeDtypeStruct((N, H), jnp.float32),
    ],
)

_tcs = pl.pallas_call(
    _tcs_body,
    grid=(GRID,),
    in_specs=[_row_spec(H), _degp_spec],
    out_specs=[_row_spec(H)],
    out_shape=[jax.ShapeDtypeStruct((N, H), jnp.float32)],
)

_tc2 = pl.pallas_call(
    _tc2_body,
    grid=(GRID,),
    in_specs=[
        _accp_spec,
        _row_spec(H),
        _degp_spec,
        _full_spec((1, H)),
        _full_spec((1, H)),
        _full_spec((1, H)),
        _full_spec((H, H)),
    ],
    out_specs=[_row_spec(H), _row_spec(H)],
    out_shape=[
        jax.ShapeDtypeStruct((N, H), jnp.float32),
        jax.ShapeDtypeStruct((N, H), jnp.float32),
    ],
)

_tc3 = pl.pallas_call(
    _tc3_body,
    grid=(GRID,),
    in_specs=[
        _accp_spec,
        _row_spec(H),
        _degp_spec,
        _full_spec((1, H)),
        _full_spec((1, H)),
        _full_spec((1, H)),
        _row_spec(H),
        _row_spec(H),
        _full_spec((H, 128)),
        _full_spec((H, 128)),
        _full_spec((H, 128)),
        _full_spec((1, 128)),
    ],
    out_specs=[_row_spec(C)],
    out_shape=[jax.ShapeDtypeStruct((N, C), jnp.float32)],
)


def kernel(x, edge_index, edge_weight, W_first, b_first, bn1_g, bn1_b,
           Wc1, bc1, bng1, bnb1, Wc2, bc2, bng2, bnb2, W_out, b_out):
    ei = edge_index.reshape(2, ROWS2D, CH)
    w2d = edge_weight.reshape(ROWS2D, CH)

    degp = _sc_deg(ei, w2d)
    degp3 = degp.reshape(2, NPAD, 1)

    wft = W_first.T
    wc1t = Wc1.T
    wc2t = Wc2.T
    row = lambda v: v.reshape(1, H)

    h0, hw1 = _tc1(x, wft, row(b_first), row(bn1_g), row(bn1_b), wc1t)
    (g1,) = _tcs(hw1, degp3)
    acc1 = _sc_conv(g1, ei, w2d)
    h1, g2 = _tc2(acc1, g1, degp3, row(bc1), row(bng1), row(bnb1), wc2t)
    acc2 = _sc_conv(g2, ei, w2d)

    wot = jnp.pad(W_out.T, ((0, 0), (0, 128 - C)))
    bpad = jnp.pad(b_out, (0, 128 - C), constant_values=-1e30).reshape(1, 128)
    (out,) = _tc3(acc2, g2, degp3, row(bc2), row(bng2), row(bnb2),
                  h0, h1, wot[:H], wot[H:2 * H], wot[2 * H:], bpad)
    return out
